# baseline (speedup 1.0000x reference)
import functools

import jax
import jax.numpy as jnp
from jax import lax
from jax.experimental import pallas as pl
from jax.experimental.pallas import tpu as pltpu
from jax.experimental.pallas import tpu_sc as plsc

_EMBED_DIM = 32
_NUM_CORES = 2
_NUM_SUBCORES = 16
_NUM_WORKERS = _NUM_CORES * _NUM_SUBCORES
_BPW = 128
_TC = 4


@functools.partial(jax.jit, static_argnames=("hist",))
def _sc_gather(idx_t, table, *, hist):
    batch = idx_t.shape[1]
    n_chunks = hist // _TC
    mesh = plsc.VectorSubcoreMesh(core_axis_name="c", subcore_axis_name="s")

    @functools.partial(
        pl.kernel,
        mesh=mesh,
        out_type=jax.ShapeDtypeStruct((hist, 4, _NUM_WORKERS, 8, 128),
                                      jnp.float32),
        scratch_types=[
            pltpu.VMEM((_TC, _BPW), jnp.int32),
            pltpu.VMEM((_TC, _BPW), jnp.int32),
            pltpu.VMEM((_TC * _BPW, _EMBED_DIM), jnp.float32),
            pltpu.VMEM((_TC * _BPW, _EMBED_DIM), jnp.float32),
            pltpu.VMEM((_TC, 4, 1, 8, 128), jnp.float32),
            pltpu.VMEM((_TC, 4, 1, 8, 128), jnp.float32),
            pltpu.SemaphoreType.DMA,
            pltpu.SemaphoreType.DMA,
            pltpu.SemaphoreType.DMA,
            pltpu.SemaphoreType.DMA,
            pltpu.SemaphoreType.DMA,
            pltpu.SemaphoreType.DMA,
        ],
        compiler_params=pltpu.CompilerParams(use_tc_tiling_on_sc=False,
                                             needs_layout_passes=False),
    )
    def k(idx_hbm, table_hbm, out_hbm,
          idx_v0, idx_v1, rows_v0, rows_v1, tile_v0, tile_v1,
          isem0, isem1, gsem0, gsem1, osem0, osem1):
        wid = lax.axis_index("s") * _NUM_CORES + lax.axis_index("c")
        b0 = wid * _BPW
        lane = lax.iota(jnp.int32, 16)
        idx_v = (idx_v0, idx_v1)
        rows_v = (rows_v0, rows_v1)
        tile_v = (tile_v0, tile_v1)
        isem = (isem0, isem1)
        gsem = (gsem0, gsem1)
        osem = (osem0, osem1)

        def idx_slice(j):
            return idx_hbm.at[pl.ds(j * _TC, _TC), pl.ds(b0, _BPW)]

        def out_slice(j):
            return out_hbm.at[pl.ds(j * _TC, _TC), pl.ds(0, 4),
                              pl.ds(wid, 1), pl.ds(0, 8), pl.ds(0, 128)]

        pltpu.async_copy(idx_slice(0), idx_v[0], isem[0])
        pltpu.async_copy(idx_slice(1), idx_v[1], isem[1])

        def do_chunk(j, b):
            pltpu.make_async_copy(idx_slice(j), idx_v[b], isem[b]).wait()
            for tp in range(_TC):
                pltpu.async_copy(
                    table_hbm.at[idx_v[b].at[tp]],
                    rows_v[b].at[pl.ds(tp * _BPW, _BPW)], gsem[b])

            @pl.when(j >= 2)
            def _():
                pltpu.make_async_copy(tile_v[b], out_slice(j - 2),
                                      osem[b]).wait()

            for tp in range(_TC):
                pltpu.make_async_copy(
                    table_hbm.at[idx_v[b].at[tp]],
                    rows_v[b].at[pl.ds(tp * _BPW, _BPW)], gsem[b]).wait()

            @pl.when(j + 2 < n_chunks)
            def _():
                pltpu.async_copy(idx_slice(j + 2), idx_v[b], isem[b])

            for tp in range(_TC):
                for e in range(_EMBED_DIM):
                    te, ei = e // 8, e % 8
                    col = jnp.full((16,), e, jnp.int32)
                    for g in range(8):
                        row = lane + (tp * _BPW + g * 16)
                        vals = plsc.load_gather(rows_v[b], [row, col])
                        tile_v[b][tp, te, 0, ei, pl.ds(g * 16, 16)] = vals

            pltpu.async_copy(tile_v[b], out_slice(j), osem[b])

        def pair(j2, carry):
            do_chunk(j2 * 2, 0)
            do_chunk(j2 * 2 + 1, 1)
            return carry

        lax.fori_loop(0, n_chunks // 2, pair, 0)
        for j in (n_chunks - 2, n_chunks - 1):
            b = j % 2
            pltpu.make_async_copy(tile_v[b], out_slice(j), osem[b]).wait()

    return k(idx_t, table)


def kernel(inputs, embedding_matrix):
    batch, hist = inputs.shape
    idx_t = inputs.T.astype(jnp.int32)
    out5 = _sc_gather(idx_t, embedding_matrix, hist=hist)
    x = out5.transpose(2, 4, 0, 1, 3)
    return x.reshape(batch, hist, _EMBED_DIM)


# speedup vs baseline: 1.0848x; 1.0848x over previous
import functools

import jax
import jax.numpy as jnp
from jax import lax
from jax.experimental import pallas as pl
from jax.experimental.pallas import tpu as pltpu
from jax.experimental.pallas import tpu_sc as plsc

_EMBED_DIM = 32
_NUM_CORES = 2
_NUM_SUBCORES = 16
_NUM_WORKERS = _NUM_CORES * _NUM_SUBCORES
_BPW = 128
_TC = 4


@functools.partial(jax.jit, static_argnames=("hist",))
def _sc_gather(idx_t, table, *, hist):
    batch = idx_t.shape[1]
    n_chunks = hist // _TC
    mesh = plsc.VectorSubcoreMesh(core_axis_name="c", subcore_axis_name="s")

    @functools.partial(
        pl.kernel,
        mesh=mesh,
        out_type=jax.ShapeDtypeStruct((hist, 4, _NUM_WORKERS, 8, 128),
                                      jnp.float32),
        scratch_types=[
            pltpu.VMEM((_TC, _BPW), jnp.int32),
            pltpu.VMEM((_TC, _BPW), jnp.int32),
            pltpu.VMEM((_TC * _BPW, _EMBED_DIM), jnp.float32),
            pltpu.VMEM((_TC * _BPW, _EMBED_DIM), jnp.float32),
            pltpu.VMEM((_TC, 4, 1, 8, 128), jnp.float32),
            pltpu.VMEM((_TC, 4, 1, 8, 128), jnp.float32),
            pltpu.SemaphoreType.DMA,
            pltpu.SemaphoreType.DMA,
            pltpu.SemaphoreType.DMA,
            pltpu.SemaphoreType.DMA,
            pltpu.SemaphoreType.DMA,
            pltpu.SemaphoreType.DMA,
        ],
        compiler_params=pltpu.CompilerParams(use_tc_tiling_on_sc=False,
                                             needs_layout_passes=False),
    )
    def k(idx_hbm, table_hbm, out_hbm,
          idx_v0, idx_v1, rows_v0, rows_v1, tile_v0, tile_v1,
          isem0, isem1, gsem0, gsem1, osem0, osem1):
        wid = lax.axis_index("s") * _NUM_CORES + lax.axis_index("c")
        b0 = wid * _BPW
        lane = lax.iota(jnp.int32, 16)
        idx_v = (idx_v0, idx_v1)
        rows_v = (rows_v0, rows_v1)
        tile_v = (tile_v0, tile_v1)
        isem = (isem0, isem1)
        gsem = (gsem0, gsem1)
        osem = (osem0, osem1)

        def idx_slice(j):
            return idx_hbm.at[pl.ds(j * _TC, _TC), pl.ds(b0, _BPW)]

        def out_slice(j):
            return out_hbm.at[pl.ds(j * _TC, _TC), pl.ds(0, 4),
                              pl.ds(wid, 1), pl.ds(0, 8), pl.ds(0, 128)]

        pltpu.async_copy(idx_slice(0), idx_v[0], isem[0])
        pltpu.async_copy(idx_slice(1), idx_v[1], isem[1])

        def do_chunk(j, b):
            pltpu.make_async_copy(idx_slice(j), idx_v[b], isem[b]).wait()
            for tp in range(_TC):
                pltpu.async_copy(
                    table_hbm.at[idx_v[b].at[tp]],
                    rows_v[b].at[pl.ds(tp * _BPW, _BPW)], gsem[b])

            @pl.when(j >= 2)
            def _():
                pltpu.make_async_copy(tile_v[b], out_slice(j - 2),
                                      osem[b]).wait()

            for tp in range(_TC):
                pltpu.make_async_copy(
                    table_hbm.at[idx_v[b].at[tp]],
                    rows_v[b].at[pl.ds(tp * _BPW, _BPW)], gsem[b]).wait()

            @pl.when(j + 2 < n_chunks)
            def _():
                pltpu.async_copy(idx_slice(j + 2), idx_v[b], isem[b])

            def trans(e, carry2):
                te = e // 8
                ei = e % 8
                col = jnp.full((16,), 0, jnp.int32) + e
                for tp in range(_TC):
                    for g in range(8):
                        row = lane + (tp * _BPW + g * 16)
                        vals = plsc.load_gather(rows_v[b], [row, col])
                        tile_v[b][tp, te, 0, ei, pl.ds(g * 16, 16)] = vals
                return carry2

            lax.fori_loop(0, _EMBED_DIM, trans, 0)
            pltpu.async_copy(tile_v[b], out_slice(j), osem[b])

        def pair(j2, carry):
            do_chunk(j2 * 2, 0)
            do_chunk(j2 * 2 + 1, 1)
            return carry

        lax.fori_loop(0, n_chunks // 2, pair, 0)
        for j in (n_chunks - 2, n_chunks - 1):
            b = j % 2
            pltpu.make_async_copy(tile_v[b], out_slice(j), osem[b]).wait()

    return k(idx_t, table)


def kernel(inputs, embedding_matrix):
    batch, hist = inputs.shape
    idx_t = inputs.T.astype(jnp.int32)
    out5 = _sc_gather(idx_t, embedding_matrix, hist=hist)
    x = out5.transpose(2, 4, 0, 1, 3)
    return x.reshape(batch, hist, _EMBED_DIM)


# speedup vs baseline: 1.3094x; 1.2071x over previous
import functools

import jax
import jax.numpy as jnp
from jax import lax
from jax.experimental import pallas as pl
from jax.experimental.pallas import tpu as pltpu
from jax.experimental.pallas import tpu_sc as plsc

_EMBED_DIM = 32
_NUM_CORES = 2
_NUM_SUBCORES = 16
_NUM_WORKERS = _NUM_CORES * _NUM_SUBCORES


@functools.partial(jax.jit, static_argnames=("chunk", "n_chunks"))
def _sc_gather(idx, table, *, chunk, n_chunks):
    b_total = idx.shape[0]
    b_per_w = b_total // _NUM_WORKERS
    mesh = plsc.VectorSubcoreMesh(core_axis_name="c", subcore_axis_name="s")

    @functools.partial(
        pl.kernel,
        mesh=mesh,
        out_type=jax.ShapeDtypeStruct((b_total, _EMBED_DIM), jnp.float32),
        scratch_types=[
            pltpu.VMEM((chunk,), jnp.int32),
            pltpu.VMEM((chunk,), jnp.int32),
            pltpu.VMEM((chunk, _EMBED_DIM), jnp.float32),
            pltpu.VMEM((chunk, _EMBED_DIM), jnp.float32),
            pltpu.SemaphoreType.DMA,
            pltpu.SemaphoreType.DMA,
            pltpu.SemaphoreType.DMA,
            pltpu.SemaphoreType.DMA,
            pltpu.SemaphoreType.DMA,
            pltpu.SemaphoreType.DMA,
        ],
        compiler_params=pltpu.CompilerParams(use_tc_tiling_on_sc=False),
    )
    def k(idx_hbm, table_hbm, out_hbm,
          idx_v0, idx_v1, rows_v0, rows_v1,
          isem0, isem1, gsem0, gsem1, osem0, osem1):
        wid = lax.axis_index("s") * _NUM_CORES + lax.axis_index("c")
        base = wid * b_per_w
        idx_v = (idx_v0, idx_v1)
        rows_v = (rows_v0, rows_v1)
        isem = (isem0, isem1)
        gsem = (gsem0, gsem1)
        osem = (osem0, osem1)

        def start_idx(j, b):
            pltpu.async_copy(
                idx_hbm.at[pl.ds(base + j * chunk, chunk)], idx_v[b], isem[b])

        start_idx(0, 0)
        if n_chunks > 1:
            start_idx(1, 1)

        for j in range(n_chunks):
            b = j % 2
            pltpu.make_async_copy(
                idx_hbm.at[pl.ds(base + j * chunk, chunk)], idx_v[b],
                isem[b]).wait()
            if j >= 2:
                pltpu.make_async_copy(
                    rows_v[b],
                    out_hbm.at[pl.ds(base + (j - 2) * chunk, chunk)],
                    osem[b]).wait()
            pltpu.async_copy(table_hbm.at[idx_v[b]], rows_v[b], gsem[b])
            pltpu.make_async_copy(
                table_hbm.at[idx_v[b]], rows_v[b], gsem[b]).wait()
            pltpu.async_copy(
                rows_v[b], out_hbm.at[pl.ds(base + j * chunk, chunk)],
                osem[b])
            if j + 2 < n_chunks:
                start_idx(j + 2, b)

        for j in (n_chunks - 2, n_chunks - 1):
            if j >= 0:
                b = j % 2
                pltpu.make_async_copy(
                    rows_v[b], out_hbm.at[pl.ds(base + j * chunk, chunk)],
                    osem[b]).wait()

    return k(idx, table)


def kernel(inputs, embedding_matrix):
    batch, hist = inputs.shape
    idx = inputs.reshape(-1)
    b_per_w = idx.shape[0] // _NUM_WORKERS
    chunk = 1600
    out = _sc_gather(idx, embedding_matrix, chunk=chunk,
                     n_chunks=b_per_w // chunk)
    return out.reshape(batch, hist, _EMBED_DIM)
